# baseline (speedup 1.0000x reference)
import functools

import jax
import jax.numpy as jnp
from jax.experimental import pallas as pl


def _fused_log1p_matmul_kernel(x_ref, emb_ref, out_ref):
    y = jnp.log2(1.0 + x_ref[...]).astype(jnp.bfloat16)
    out_ref[...] = jax.lax.dot_general(
        y,
        emb_ref[...],
        dimension_numbers=(((1,), (0,)), ((), ())),
        preferred_element_type=jnp.float32,
    )


@jax.jit
def kernel(x_seq, emb):
    B, P = x_seq.shape
    H = emb.shape[1]
    block_b = 8192
    grid = (B // block_b,)
    emb_scaled = (emb * jnp.float32(0.6931471805599453)).astype(jnp.bfloat16)
    return pl.pallas_call(
        _fused_log1p_matmul_kernel,
        grid=grid,
        in_specs=[
            pl.BlockSpec((block_b, P), lambda i: (i, 0)),
            pl.BlockSpec((P, H), lambda i: (0, 0)),
        ],
        out_specs=pl.BlockSpec((block_b, H), lambda i: (i, 0)),
        out_shape=jax.ShapeDtypeStruct((B, H), jnp.float32),
    )(x_seq, emb_scaled)


# speedup vs baseline: 1.0553x; 1.0553x over previous
import functools

import jax
import jax.numpy as jnp
from jax.experimental import pallas as pl


def _fused_log1p_matmul_kernel(x_ref, emb_ref, out_ref):
    y = jnp.log2(1.0 + x_ref[...]).astype(jnp.bfloat16)
    out_ref[...] = jax.lax.dot_general(
        y,
        emb_ref[...],
        dimension_numbers=(((1,), (0,)), ((), ())),
        preferred_element_type=jnp.float32,
    )


@jax.jit
def kernel(x_seq, emb):
    B, P = x_seq.shape
    H = emb.shape[1]
    block_b = 4096
    grid = (B // block_b,)
    emb_scaled = (emb * jnp.float32(0.6931471805599453)).astype(jnp.bfloat16)
    return pl.pallas_call(
        _fused_log1p_matmul_kernel,
        grid=grid,
        in_specs=[
            pl.BlockSpec((block_b, P), lambda i: (i, 0)),
            pl.BlockSpec((P, H), lambda i: (0, 0)),
        ],
        out_specs=pl.BlockSpec((block_b, H), lambda i: (i, 0)),
        out_shape=jax.ShapeDtypeStruct((B, H), jnp.float32),
    )(x_seq, emb_scaled)


# speedup vs baseline: 1.1612x; 1.1004x over previous
import functools

import jax
import jax.numpy as jnp
from jax.experimental import pallas as pl

_LN2 = 0.6931471805599453


def _fused_log1p_matmul_kernel(xa_ref, xb_ref, emb_ref, out_ref):
    feat = (emb_ref[...] * _LN2).astype(jnp.bfloat16)
    ya = jnp.log2(1.0 + xa_ref[0]).astype(jnp.bfloat16)
    yb = jnp.log2(1.0 + xb_ref[0]).astype(jnp.bfloat16)
    dims = (((1,), (0,)), ((), ()))
    out_ref[0] = jax.lax.dot_general(
        ya, feat, dimension_numbers=dims, preferred_element_type=jnp.float32
    )
    out_ref[1] = jax.lax.dot_general(
        yb, feat, dimension_numbers=dims, preferred_element_type=jnp.float32
    )


@jax.jit
def kernel(x_seq, emb):
    B, P = x_seq.shape
    H = emb.shape[1]
    block_b = 2048
    n_chunks = B // block_b
    x3 = x_seq.reshape(n_chunks, block_b, P)
    out3 = pl.pallas_call(
        _fused_log1p_matmul_kernel,
        grid=(n_chunks // 2,),
        in_specs=[
            pl.BlockSpec((1, block_b, P), lambda i: (2 * i, 0, 0)),
            pl.BlockSpec((1, block_b, P), lambda i: (2 * i + 1, 0, 0)),
            pl.BlockSpec((P, H), lambda i: (0, 0)),
        ],
        out_specs=pl.BlockSpec((2, block_b, H), lambda i: (i, 0, 0)),
        out_shape=jax.ShapeDtypeStruct((n_chunks, block_b, H), jnp.float32),
    )(x3, x3, emb)
    return out3.reshape(B, H)
